# baseline (speedup 1.0000x reference)
import functools

import jax
import jax.numpy as jnp
from jax import lax
from jax.experimental import pallas as pl
from jax.experimental.pallas import tpu as pltpu
from jax.experimental.pallas import tpu_sc as plsc

N_FINE = 500000
N_COARSE = 100000
DIM = 128
LANES = 16

C = 128
N_CHUNKS = N_FINE // C
TAIL = N_FINE - N_CHUNKS * C
TAIL_START = N_CHUNKS * C
NUM_WORKERS = 32
NBUF = 3

Q = N_CHUNKS // NUM_WORKERS
R = N_CHUNKS % NUM_WORKERS
MAX_W_CHUNKS = Q + 1
ARENA_CHUNKS = MAX_W_CHUNKS + 1
J_MAX = -(-(MAX_W_CHUNKS + 1) // NBUF) * NBUF


def _sc_kernel(x_hbm, up_hbm, inv_hbm, out_hbm,
               arena, rows0, rows1, rows2, upb0, upb1, upb2,
               sg0, sg1, sg2, su0, su1, su2, so0, so1, so2):
    rows = (rows0, rows1, rows2)
    upb = (upb0, upb1, upb2)
    sg = (sg0, sg1, sg2)
    su = (su0, su1, su2)
    so = (so0, so1, so2)
    wid = lax.axis_index("s") * 2 + lax.axis_index("c")

    n_w = Q + (wid < R).astype(jnp.int32)
    base = wid * Q + lax.min(wid, R)

    pltpu.sync_copy(inv_hbm.at[pl.ds(base * C, Q * C)],
                    arena.at[pl.ds(0, Q * C)])

    @pl.when(wid < R)
    def _():
        pltpu.sync_copy(inv_hbm.at[pl.ds((base + Q) * C, C)],
                        arena.at[pl.ds(Q * C, C)])

    def exists(j):
        return j < n_w

    def start_of(j):
        return (base + j) * C

    def issue_loads(j, b):
        @pl.when(exists(j))
        def _():
            @pl.when(j >= NBUF)
            def _():
                pltpu.make_async_copy(
                    rows[b], out_hbm.at[pl.ds(0, C)], so[b]).wait()

            pltpu.async_copy(x_hbm.at[arena.at[pl.ds(j * C, C)]],
                             rows[b], sg[b])
            pltpu.async_copy(up_hbm.at[pl.ds(start_of(j), C)], upb[b], su[b])

    def compute_write(j, b):
        @pl.when(exists(j))
        def _():
            pltpu.make_async_copy(x_hbm.at[arena.at[pl.ds(0, C)]],
                                  rows[b], sg[b]).wait()
            pltpu.make_async_copy(up_hbm.at[pl.ds(0, C)], upb[b], su[b]).wait()

            @plsc.parallel_loop(0, C, step=1, unroll=4)
            def _add_row(r):
                for l in range(DIM // LANES):
                    sl = pl.ds(l * LANES, LANES)
                    plsc.addupdate(rows[b].at[r, sl], upb[b][r, sl])

            pltpu.async_copy(rows[b], out_hbm.at[pl.ds(start_of(j), C)], so[b])

    issue_loads(0, 0)

    def triple_body(i, _):
        for bb in range(NBUF):
            j = 1 + NBUF * i + bb
            b = (1 + bb) % NBUF
            issue_loads(j, b)
            compute_write(j - 1, bb)
        return 0

    lax.fori_loop(0, J_MAX // NBUF, triple_body, 0)

    for b in range(NBUF):
        pltpu.make_async_copy(rows[b], out_hbm.at[pl.ds(0, C)], so[b]).wait()

    if TAIL:
        @pl.when(wid == 0)
        def _():
            pltpu.sync_copy(inv_hbm.at[pl.ds(TAIL_START, TAIL)],
                            arena.at[pl.ds(MAX_W_CHUNKS * C, TAIL)])
            pltpu.async_copy(
                x_hbm.at[arena.at[pl.ds(MAX_W_CHUNKS * C, TAIL)]],
                rows[0].at[pl.ds(0, TAIL)], sg[0]).wait()
            pltpu.sync_copy(up_hbm.at[pl.ds(TAIL_START, TAIL)],
                            upb[0].at[pl.ds(0, TAIL)])

            @plsc.parallel_loop(0, TAIL, step=1, unroll=4)
            def _add_row_tail(r):
                for l in range(DIM // LANES):
                    sl = pl.ds(l * LANES, LANES)
                    plsc.addupdate(rows[0].at[r, sl], upb[0][r, sl])

            pltpu.sync_copy(rows[0].at[pl.ds(0, TAIL)],
                            out_hbm.at[pl.ds(TAIL_START, TAIL)])


@jax.jit
def _run(x_features, up_x_features, unq_inv):
    mesh = plsc.VectorSubcoreMesh(core_axis_name="c", subcore_axis_name="s")
    return pl.kernel(
        _sc_kernel,
        mesh=mesh,
        out_type=jax.ShapeDtypeStruct((N_FINE, DIM), jnp.float32),
        scratch_types=(
            [pltpu.VMEM((ARENA_CHUNKS * C,), jnp.int32)]
            + [pltpu.VMEM((C, DIM), jnp.float32)] * (2 * NBUF)
            + [pltpu.SemaphoreType.DMA] * (3 * NBUF)
        ),
    )(x_features, up_x_features, unq_inv)


def kernel(x_features, up_x_features, unq_inv):
    return _run(x_features, up_x_features, unq_inv.astype(jnp.int32))


# speedup vs baseline: 1.0343x; 1.0343x over previous
import functools

import jax
import jax.numpy as jnp
from jax import lax
from jax.experimental import pallas as pl
from jax.experimental.pallas import tpu as pltpu
from jax.experimental.pallas import tpu_sc as plsc

N_FINE = 500000
N_COARSE = 100000
DIM = 128
LANES = 16

C = 96
N_CHUNKS = N_FINE // C
TAIL = N_FINE - N_CHUNKS * C
TAIL_START = N_CHUNKS * C
NUM_WORKERS = 32
NSC_TILES = 16

Q = N_CHUNKS // NUM_WORKERS
R = N_CHUNKS % NUM_WORKERS
MAX_W_CHUNKS = Q + 1
ARENA_CHUNKS = MAX_W_CHUNKS + 1
J_MAX = -(-(MAX_W_CHUNKS + 2) // 4) * 4


def _sc_kernel(x_hbm, up_hbm, inv_hbm, out_hbm,
               arena, iota_v, rows_t0, rows_t1, up_t0, up_t1,
               rows_s0, rows_s1, slab,
               sgt0, sgt1, sut0, sut1, sot0, sot1,
               sgs0, sgs1, sus0, sus1, sos0, sos1):
    rows_t = (rows_t0, rows_t1)
    up_t = (up_t0, up_t1)
    rows_s = (rows_s0, rows_s1)
    sgt = (sgt0, sgt1)
    sut = (sut0, sut1)
    sot = (sot0, sot1)
    sgs = (sgs0, sgs1)
    sus = (sus0, sus1)
    sos = (sos0, sos1)

    sid = lax.axis_index("s")
    wid = sid * 2 + lax.axis_index("c")

    n_w = Q + (wid < R).astype(jnp.int32)
    base = wid * Q + lax.min(wid, R)

    for l in range(C // LANES):
        iota_v[pl.ds(l * LANES, LANES)] = lax.iota(jnp.int32, LANES) + l * LANES

    pltpu.sync_copy(inv_hbm.at[pl.ds(base * C, Q * C)],
                    arena.at[pl.ds(0, Q * C)])

    @pl.when(wid < R)
    def _():
        pltpu.sync_copy(inv_hbm.at[pl.ds((base + Q) * C, C)],
                        arena.at[pl.ds(Q * C, C)])

    def exists(j):
        return j < n_w

    def start_of(j):
        return (base + j) * C

    def t_issue(j, t):
        @pl.when(exists(j))
        def _():
            @pl.when(j >= 4)
            def _():
                pltpu.make_async_copy(
                    rows_t[t], out_hbm.at[pl.ds(0, C)], sot[t]).wait()

            pltpu.async_copy(x_hbm.at[arena.at[pl.ds(j * C, C)]],
                             rows_t[t], sgt[t])
            pltpu.async_copy(up_hbm.at[pl.ds(start_of(j), C)], up_t[t], sut[t])

    def t_finish(j, t):
        @pl.when(exists(j))
        def _():
            pltpu.make_async_copy(x_hbm.at[arena.at[pl.ds(0, C)]],
                                  rows_t[t], sgt[t]).wait()
            pltpu.make_async_copy(up_hbm.at[pl.ds(0, C)],
                                  up_t[t], sut[t]).wait()

            @plsc.parallel_loop(0, C, step=1, unroll=4)
            def _add_row(r):
                for l in range(DIM // LANES):
                    sl = pl.ds(l * LANES, LANES)
                    plsc.addupdate(rows_t[t].at[r, sl], up_t[t][r, sl])

            pltpu.async_copy(rows_t[t],
                             out_hbm.at[pl.ds(start_of(j), C)], sot[t])

    def s_issue(j, s):
        @pl.when(exists(j))
        def _():
            @pl.when(j >= 4)
            def _():
                pltpu.make_async_copy(
                    slab.at[sid, s], out_hbm.at[pl.ds(0, C)], sos[s]).wait()

            pltpu.async_copy(up_hbm.at[pl.ds(start_of(j), C)],
                             slab.at[sid, s], sus[s])
            pltpu.async_copy(x_hbm.at[arena.at[pl.ds(j * C, C)]],
                             rows_s[s], sgs[s])

    def s_finish(j, s):
        @pl.when(exists(j))
        def _():
            pltpu.make_async_copy(x_hbm.at[arena.at[pl.ds(0, C)]],
                                  rows_s[s], sgs[s]).wait()
            pltpu.make_async_copy(up_hbm.at[pl.ds(0, C)],
                                  slab.at[sid, s], sus[s]).wait()
            pltpu.sync_copy(rows_s[s], slab.at[sid, s].at[iota_v], add=True)
            pltpu.async_copy(slab.at[sid, s],
                             out_hbm.at[pl.ds(start_of(j), C)], sos[s])

    t_issue(0, 0)
    s_issue(1, 0)

    def quad_body(i, _):
        for bb in range(4):
            j = 2 + 4 * i + bb
            slot_j = (1 + bb // 2) % 2
            slot_p = (bb // 2) % 2
            if bb % 2 == 0:
                t_issue(j, slot_j)
                t_finish(j - 2, slot_p)
            else:
                s_issue(j, slot_j)
                s_finish(j - 2, slot_p)
        return 0

    lax.fori_loop(0, (J_MAX - 2) // 4, quad_body, 0)

    for t in range(2):
        pltpu.make_async_copy(rows_t[t], out_hbm.at[pl.ds(0, C)], sot[t]).wait()
    for s in range(2):
        pltpu.make_async_copy(
            slab.at[sid, s], out_hbm.at[pl.ds(0, C)], sos[s]).wait()

    if TAIL:
        @pl.when(wid == 0)
        def _():
            pltpu.sync_copy(inv_hbm.at[pl.ds(TAIL_START, TAIL)],
                            arena.at[pl.ds(MAX_W_CHUNKS * C, TAIL)])
            pltpu.async_copy(
                x_hbm.at[arena.at[pl.ds(MAX_W_CHUNKS * C, TAIL)]],
                rows_t[0].at[pl.ds(0, TAIL)], sgt[0]).wait()
            pltpu.sync_copy(up_hbm.at[pl.ds(TAIL_START, TAIL)],
                            up_t[0].at[pl.ds(0, TAIL)])

            @plsc.parallel_loop(0, TAIL, step=1, unroll=4)
            def _add_row_tail(r):
                for l in range(DIM // LANES):
                    sl = pl.ds(l * LANES, LANES)
                    plsc.addupdate(rows_t[0].at[r, sl], up_t[0][r, sl])

            pltpu.sync_copy(rows_t[0].at[pl.ds(0, TAIL)],
                            out_hbm.at[pl.ds(TAIL_START, TAIL)])


@jax.jit
def _run(x_features, up_x_features, unq_inv):
    mesh = plsc.VectorSubcoreMesh(core_axis_name="c", subcore_axis_name="s")
    return pl.kernel(
        _sc_kernel,
        mesh=mesh,
        out_type=jax.ShapeDtypeStruct((N_FINE, DIM), jnp.float32),
        scratch_types=(
            [pltpu.VMEM((ARENA_CHUNKS * C,), jnp.int32),
             pltpu.VMEM((C,), jnp.int32)]
            + [pltpu.VMEM((C, DIM), jnp.float32)] * 6
            + [pltpu.VMEM_SHARED((NSC_TILES, 2, C, DIM), jnp.float32)]
            + [pltpu.SemaphoreType.DMA] * 12
        ),
    )(x_features, up_x_features, unq_inv)


def kernel(x_features, up_x_features, unq_inv):
    return _run(x_features, up_x_features, unq_inv.astype(jnp.int32))


# speedup vs baseline: 1.0394x; 1.0050x over previous
import functools

import jax
import jax.numpy as jnp
from jax import lax
from jax.experimental import pallas as pl
from jax.experimental.pallas import tpu as pltpu
from jax.experimental.pallas import tpu_sc as plsc

N_FINE = 500000
N_COARSE = 100000
DIM = 128
LANES = 16

C = 96
N_CHUNKS = N_FINE // C
TAIL = N_FINE - N_CHUNKS * C
TAIL_START = N_CHUNKS * C
NUM_WORKERS = 32
NSC_TILES = 16

Q = N_CHUNKS // NUM_WORKERS
R = N_CHUNKS % NUM_WORKERS
MAX_W_CHUNKS = Q + 1
ARENA_CHUNKS = MAX_W_CHUNKS + 1
J_MAX = -(-(MAX_W_CHUNKS + 2) // 4) * 4


def _sc_kernel(x_hbm, up_hbm, inv_hbm, out_hbm,
               arena, iota_v, rows_t0, rows_t1, up_t0, up_t1,
               rows_s0, rows_s1, slab,
               sgt0, sgt1, sut0, sut1, sot0, sot1,
               sgs0, sgs1, sus0, sus1, sos0, sos1):
    rows_t = (rows_t0, rows_t1)
    up_t = (up_t0, up_t1)
    rows_s = (rows_s0, rows_s1)
    sgt = (sgt0, sgt1)
    sut = (sut0, sut1)
    sot = (sot0, sot1)
    sgs = (sgs0, sgs1)
    sus = (sus0, sus1)
    sos = (sos0, sos1)

    sid = lax.axis_index("s")
    wid = sid * 2 + lax.axis_index("c")

    n_w = Q + (wid < R).astype(jnp.int32)
    base = wid * Q + lax.min(wid, R)

    for l in range(C // LANES):
        iota_v[pl.ds(l * LANES, LANES)] = lax.iota(jnp.int32, LANES) + l * LANES

    pltpu.sync_copy(inv_hbm.at[pl.ds(base * C, Q * C)],
                    arena.at[pl.ds(0, Q * C)])

    @pl.when(wid < R)
    def _():
        pltpu.sync_copy(inv_hbm.at[pl.ds((base + Q) * C, C)],
                        arena.at[pl.ds(Q * C, C)])

    def exists(j):
        return j < n_w

    def start_of(j):
        return (base + j) * C

    def t_issue(j, t):
        @pl.when(exists(j))
        def _():
            @pl.when(j >= 4)
            def _():
                pltpu.make_async_copy(
                    rows_t[t], out_hbm.at[pl.ds(0, C)], sot[t]).wait()

            pltpu.async_copy(x_hbm.at[arena.at[pl.ds(j * C, C)]],
                             rows_t[t], sgt[t])
            pltpu.async_copy(up_hbm.at[pl.ds(start_of(j), C)], up_t[t], sut[t])

    def t_finish(j, t):
        @pl.when(exists(j))
        def _():
            pltpu.make_async_copy(x_hbm.at[arena.at[pl.ds(0, C)]],
                                  rows_t[t], sgt[t]).wait()
            pltpu.make_async_copy(up_hbm.at[pl.ds(0, C)],
                                  up_t[t], sut[t]).wait()

            @plsc.parallel_loop(0, C, step=1, unroll=4)
            def _add_row(r):
                for l in range(DIM // LANES):
                    sl = pl.ds(l * LANES, LANES)
                    plsc.addupdate(rows_t[t].at[r, sl], up_t[t][r, sl])

            pltpu.async_copy(rows_t[t],
                             out_hbm.at[pl.ds(start_of(j), C)], sot[t])

    def s_issue(j, s):
        @pl.when(exists(j))
        def _():
            @pl.when(j >= 4)
            def _():
                pltpu.make_async_copy(
                    slab.at[sid, s], out_hbm.at[pl.ds(0, C)], sos[s]).wait()

            pltpu.async_copy(up_hbm.at[pl.ds(start_of(j), C)],
                             slab.at[sid, s], sus[s])
            pltpu.async_copy(x_hbm.at[arena.at[pl.ds(j * C, C)]],
                             rows_s[s], sgs[s])

    def s_finish(j, s):
        @pl.when(exists(j))
        def _():
            pltpu.make_async_copy(x_hbm.at[arena.at[pl.ds(0, C)]],
                                  rows_s[s], sgs[s]).wait()
            pltpu.make_async_copy(up_hbm.at[pl.ds(0, C)],
                                  slab.at[sid, s], sus[s]).wait()
            pltpu.sync_copy(rows_s[s], slab.at[sid, s].at[iota_v], add=True)
            pltpu.async_copy(slab.at[sid, s],
                             out_hbm.at[pl.ds(start_of(j), C)], sos[s])

    t_issue(0, 0)
    s_issue(1, 0)

    def quad_body(i, _):
        for bb in range(4):
            j = 2 + 4 * i + bb
            slot_j = (1 + bb // 2) % 2
            slot_p = (bb // 2) % 2
            if bb % 2 == 0:
                t_issue(j, slot_j)
                t_finish(j - 2, slot_p)
            else:
                s_issue(j, slot_j)
                s_finish(j - 2, slot_p)
        return 0

    lax.fori_loop(0, (J_MAX - 2) // 4, quad_body, 0)

    for t in range(2):
        pltpu.make_async_copy(rows_t[t], out_hbm.at[pl.ds(0, C)], sot[t]).wait()
    for s in range(2):
        pltpu.make_async_copy(
            slab.at[sid, s], out_hbm.at[pl.ds(0, C)], sos[s]).wait()

    if TAIL:
        TPW = 8
        NTW = TAIL // TPW

        @pl.when((wid >= R) & (wid < R + NTW))
        def _():
            ts = TAIL_START + (wid - R) * TPW
            spare = MAX_W_CHUNKS * C
            pltpu.sync_copy(inv_hbm.at[pl.ds(ts, TPW)],
                            arena.at[pl.ds(spare, TPW)])
            pltpu.async_copy(
                x_hbm.at[arena.at[pl.ds(spare, TPW)]],
                rows_t[0].at[pl.ds(0, TPW)], sgt[0]).wait()
            pltpu.sync_copy(up_hbm.at[pl.ds(ts, TPW)],
                            up_t[0].at[pl.ds(0, TPW)])

            @plsc.parallel_loop(0, TPW, step=1, unroll=4)
            def _add_row_tail(r):
                for l in range(DIM // LANES):
                    sl = pl.ds(l * LANES, LANES)
                    plsc.addupdate(rows_t[0].at[r, sl], up_t[0][r, sl])

            pltpu.sync_copy(rows_t[0].at[pl.ds(0, TPW)],
                            out_hbm.at[pl.ds(ts, TPW)])


@jax.jit
def _run(x_features, up_x_features, unq_inv):
    mesh = plsc.VectorSubcoreMesh(core_axis_name="c", subcore_axis_name="s")
    return pl.kernel(
        _sc_kernel,
        mesh=mesh,
        out_type=jax.ShapeDtypeStruct((N_FINE, DIM), jnp.float32),
        scratch_types=(
            [pltpu.VMEM((ARENA_CHUNKS * C,), jnp.int32),
             pltpu.VMEM((C,), jnp.int32)]
            + [pltpu.VMEM((C, DIM), jnp.float32)] * 6
            + [pltpu.VMEM_SHARED((NSC_TILES, 2, C, DIM), jnp.float32)]
            + [pltpu.SemaphoreType.DMA] * 12
        ),
    )(x_features, up_x_features, unq_inv)


def kernel(x_features, up_x_features, unq_inv):
    return _run(x_features, up_x_features, unq_inv.astype(jnp.int32))


# speedup vs baseline: 1.0482x; 1.0085x over previous
import functools

import jax
import jax.numpy as jnp
from jax import lax
from jax.experimental import pallas as pl
from jax.experimental.pallas import tpu as pltpu
from jax.experimental.pallas import tpu_sc as plsc

N_FINE = 500000
N_COARSE = 100000
DIM = 128
LANES = 16

C = 96
N_CHUNKS = N_FINE // C
TAIL = N_FINE - N_CHUNKS * C
TAIL_START = N_CHUNKS * C
NUM_WORKERS = 32
NSC_TILES = 16

Q = N_CHUNKS // NUM_WORKERS
R = N_CHUNKS % NUM_WORKERS
MAX_W_CHUNKS = Q + 1
ARENA_CHUNKS = MAX_W_CHUNKS + 1
N_SIX = -(-(MAX_W_CHUNKS + 2 - 2) // 6)


def _sc_kernel(x_hbm, up_hbm, inv_hbm, out_hbm,
               arena, iota_v, rows_t0, rows_t1, up_t0, up_t1,
               rows_s0, rows_s1, slab,
               sgt0, sgt1, sut0, sut1, sot0, sot1,
               sgs0, sgs1, sus0, sus1, sos0, sos1):
    rows_t = (rows_t0, rows_t1)
    up_t = (up_t0, up_t1)
    rows_s = (rows_s0, rows_s1)
    sgt = (sgt0, sgt1)
    sut = (sut0, sut1)
    sot = (sot0, sot1)
    sgs = (sgs0, sgs1)
    sus = (sus0, sus1)
    sos = (sos0, sos1)

    sid = lax.axis_index("s")
    wid = sid * 2 + lax.axis_index("c")

    n_w = Q + (wid < R).astype(jnp.int32)
    base = wid * Q + lax.min(wid, R)

    for l in range(C // LANES):
        iota_v[pl.ds(l * LANES, LANES)] = lax.iota(jnp.int32, LANES) + l * LANES

    pltpu.sync_copy(inv_hbm.at[pl.ds(base * C, Q * C)],
                    arena.at[pl.ds(0, Q * C)])

    @pl.when(wid < R)
    def _():
        pltpu.sync_copy(inv_hbm.at[pl.ds((base + Q) * C, C)],
                        arena.at[pl.ds(Q * C, C)])

    def exists(j):
        return j < n_w

    def start_of(j):
        return (base + j) * C

    def t_issue(j, t):
        @pl.when(exists(j))
        def _():
            @pl.when(j >= 4)
            def _():
                pltpu.make_async_copy(
                    rows_t[t], out_hbm.at[pl.ds(0, C)], sot[t]).wait()

            pltpu.async_copy(x_hbm.at[arena.at[pl.ds(j * C, C)]],
                             rows_t[t], sgt[t])
            pltpu.async_copy(up_hbm.at[pl.ds(start_of(j), C)], up_t[t], sut[t])

    def t_finish(j, t):
        @pl.when(exists(j))
        def _():
            pltpu.make_async_copy(x_hbm.at[arena.at[pl.ds(0, C)]],
                                  rows_t[t], sgt[t]).wait()
            pltpu.make_async_copy(up_hbm.at[pl.ds(0, C)],
                                  up_t[t], sut[t]).wait()

            @plsc.parallel_loop(0, C, step=1, unroll=4)
            def _add_row(r):
                for l in range(DIM // LANES):
                    sl = pl.ds(l * LANES, LANES)
                    plsc.addupdate(rows_t[t].at[r, sl], up_t[t][r, sl])

            pltpu.async_copy(rows_t[t],
                             out_hbm.at[pl.ds(start_of(j), C)], sot[t])

    def s_issue(j, s):
        @pl.when(exists(j))
        def _():
            @pl.when(j >= 4)
            def _():
                pltpu.make_async_copy(
                    slab.at[sid, s], out_hbm.at[pl.ds(0, C)], sos[s]).wait()

            pltpu.async_copy(up_hbm.at[pl.ds(start_of(j), C)],
                             slab.at[sid, s], sus[s])
            pltpu.async_copy(x_hbm.at[arena.at[pl.ds(j * C, C)]],
                             rows_s[s], sgs[s])

    def s_finish(j, s):
        @pl.when(exists(j))
        def _():
            pltpu.make_async_copy(x_hbm.at[arena.at[pl.ds(0, C)]],
                                  rows_s[s], sgs[s]).wait()
            pltpu.make_async_copy(up_hbm.at[pl.ds(0, C)],
                                  slab.at[sid, s], sus[s]).wait()
            pltpu.sync_copy(rows_s[s], slab.at[sid, s].at[iota_v], add=True)
            pltpu.async_copy(slab.at[sid, s],
                             out_hbm.at[pl.ds(start_of(j), C)], sos[s])

    t_issue(0, 0)
    s_issue(1, 0)

    def six_body(i, _):
        for bb in range(6):
            j = 2 + 6 * i + bb
            m = (2 + bb) % 3
            if m == 0:
                slot_j = (1 + (bb - 1) // 3) % 2
                t_issue(j, slot_j)
                t_finish(j - 3, 1 - slot_j)
            elif m == 1:
                s_issue(j, 0)
                s_finish(j - 2, 1)
            else:
                s_issue(j, 1)
                s_finish(j - 1, 0)
        return 0

    lax.fori_loop(0, N_SIX, six_body, 0)

    for t in range(2):
        pltpu.make_async_copy(rows_t[t], out_hbm.at[pl.ds(0, C)], sot[t]).wait()
    for s in range(2):
        pltpu.make_async_copy(
            slab.at[sid, s], out_hbm.at[pl.ds(0, C)], sos[s]).wait()

    if TAIL:
        TPW = 8
        NTW = TAIL // TPW

        @pl.when((wid >= R) & (wid < R + NTW))
        def _():
            ts = TAIL_START + (wid - R) * TPW
            spare = MAX_W_CHUNKS * C
            pltpu.sync_copy(inv_hbm.at[pl.ds(ts, TPW)],
                            arena.at[pl.ds(spare, TPW)])
            pltpu.async_copy(
                x_hbm.at[arena.at[pl.ds(spare, TPW)]],
                rows_t[0].at[pl.ds(0, TPW)], sgt[0]).wait()
            pltpu.sync_copy(up_hbm.at[pl.ds(ts, TPW)],
                            up_t[0].at[pl.ds(0, TPW)])

            @plsc.parallel_loop(0, TPW, step=1, unroll=4)
            def _add_row_tail(r):
                for l in range(DIM // LANES):
                    sl = pl.ds(l * LANES, LANES)
                    plsc.addupdate(rows_t[0].at[r, sl], up_t[0][r, sl])

            pltpu.sync_copy(rows_t[0].at[pl.ds(0, TPW)],
                            out_hbm.at[pl.ds(ts, TPW)])


@jax.jit
def _run(x_features, up_x_features, unq_inv):
    mesh = plsc.VectorSubcoreMesh(core_axis_name="c", subcore_axis_name="s")
    return pl.kernel(
        _sc_kernel,
        mesh=mesh,
        out_type=jax.ShapeDtypeStruct((N_FINE, DIM), jnp.float32),
        scratch_types=(
            [pltpu.VMEM((ARENA_CHUNKS * C,), jnp.int32),
             pltpu.VMEM((C,), jnp.int32)]
            + [pltpu.VMEM((C, DIM), jnp.float32)] * 6
            + [pltpu.VMEM_SHARED((NSC_TILES, 2, C, DIM), jnp.float32)]
            + [pltpu.SemaphoreType.DMA] * 12
        ),
    )(x_features, up_x_features, unq_inv)


def kernel(x_features, up_x_features, unq_inv):
    return _run(x_features, up_x_features, unq_inv.astype(jnp.int32))
